# baseline (speedup 1.0000x reference)
import functools

import jax
import jax.numpy as jnp
from jax import lax
from jax.experimental import pallas as pl
from jax.experimental.pallas import tpu as pltpu
from jax.experimental.pallas import tpu_sc as plsc

N = 10000
E = 320000
D = 128
L = 16
NC = 2
NS = 16
NW = NC * NS
K = 48
EPT = 10368
E2 = NW * EPT
NCH = EPT // K
NB = 4
RSTRIDE = 624
RPT = 640


def _lane_bcast(vec, k):
    idx = jnp.full((L, 1), k, dtype=jnp.int32)
    return lax.gather(
        vec, idx,
        lax.GatherDimensionNumbers(
            offset_dims=(), collapsed_slice_dims=(0,), start_index_map=(0,)),
        (1,), mode=lax.GatherScatterMode.PROMISE_IN_BOUNDS)


def _spmm_body(h_hbm, src_hbm, dst_hbm, w_hbm, out_hbm,
               src_v, w_v, dst_bufs, row_bufs, acc_sh,
               semg, semd, sems):
    c = lax.axis_index("c")
    s = lax.axis_index("s")
    wid = c * NS + s
    ebase = wid * EPT

    pltpu.sync_copy(src_hbm.at[pl.ds(ebase, EPT)], src_v)
    pltpu.sync_copy(w_hbm.at[pl.ds(ebase, EPT)], w_v)

    zero16 = jnp.zeros((L,), jnp.float32)
    def zfill(i, _):
        for j in range(D // L):
            row_bufs[0][i, pl.ds(j * L, L)] = zero16
            row_bufs[1][i, pl.ds(j * L, L)] = zero16
        return 0
    lax.fori_loop(0, K, zfill, 0)
    row0 = s * RSTRIDE
    nz = RPT // K
    for z in range(nz):
        pltpu.sync_copy(row_bufs[z % 2], acc_sh.at[pl.ds(row0 + z * K, K)])
    if RPT % K:
        pltpu.sync_copy(row_bufs[nz % 2],
                        acc_sh.at[pl.ds(row0 + RPT - K, K)])
    plsc.subcore_barrier()

    def issue(i, u):
        pltpu.async_copy(h_hbm.at[src_v.at[pl.ds(i * K, K)]],
                         row_bufs[u], semg[u])
        pltpu.async_copy(dst_hbm.at[pl.ds(ebase + i * K, K)],
                         dst_bufs[u], semd[u])

    def mult(i, u):
        pltpu.make_async_copy(h_hbm.at[src_v.at[pl.ds(i * K, K)]],
                              row_bufs[u], semg[u]).wait()
        pltpu.make_async_copy(dst_hbm.at[pl.ds(ebase + i * K, K)],
                              dst_bufs[u], semd[u]).wait()
        rbuf = row_bufs[u]
        def group(g, _):
            wg = w_v[pl.ds(i * K + g * L, L)]
            for k in range(L):
                wk = _lane_bcast(wg, k)
                r = g * L + k
                for j in range(D // L):
                    rbuf[r, pl.ds(j * L, L)] = rbuf[r, pl.ds(j * L, L)] * wk
            return 0
        lax.fori_loop(0, K // L, group, 0)

    def scatter_start(u):
        pltpu.async_copy(row_bufs[u], acc_sh.at[dst_bufs[u]], sems[u],
                         add=True)

    def scatter_wait(u):
        pltpu.make_async_copy(row_bufs[u], acc_sh.at[dst_bufs[u]],
                              sems[u]).wait()

    issue(0, 0)
    issue(1, 1)
    mult(0, 0)
    scatter_start(0)
    issue(2, 2)
    mult(1, 1)
    scatter_start(1)
    issue(3, 3)

    def body(t, _):
        i0 = 4 * t + 2
        for d in range(NB):
            i = i0 + d
            u = (2 + d) % NB
            mult(i, u)
            scatter_start(u)
            scatter_wait((u + 2) % NB)
            issue(i + 2, (u + 2) % NB)
        return 0

    lax.fori_loop(0, (NCH - 4) // 4, body, 0)

    i0 = NCH - 2
    mult(i0, i0 % NB)
    scatter_start(i0 % NB)
    scatter_wait((i0 + 2) % NB)
    mult(i0 + 1, (i0 + 1) % NB)
    scatter_start((i0 + 1) % NB)
    scatter_wait((i0 + 3) % NB)
    scatter_wait(i0 % NB)
    scatter_wait((i0 + 1) % NB)

    plsc.subcore_barrier()

    pltpu.sync_copy(acc_sh.at[pl.ds(row0, RPT)],
                    out_hbm.at[c, pl.ds(row0, RPT)])


_spmm = pl.kernel(
    _spmm_body,
    out_type=jax.ShapeDtypeStruct((NC, N, D), jnp.float32),
    mesh=plsc.VectorSubcoreMesh(core_axis_name="c", subcore_axis_name="s"),
    scratch_types=[
        pltpu.VMEM((EPT,), jnp.int32),
        pltpu.VMEM((EPT,), jnp.float32),
        [pltpu.VMEM((K,), jnp.int32) for _ in range(NB)],
        [pltpu.VMEM((K, D), jnp.float32) for _ in range(NB)],
        pltpu.VMEM_SHARED((N, D), jnp.float32),
        [pltpu.SemaphoreType.DMA for _ in range(NB)],
        [pltpu.SemaphoreType.DMA for _ in range(NB)],
        [pltpu.SemaphoreType.DMA for _ in range(NB)],
    ],
)


_BM = 2000


def _lin1_body(x_ref, wt_ref, b_ref, o_ref):
    o_ref[...] = (jnp.dot(x_ref[...], wt_ref[...],
                          preferred_element_type=jnp.float32)
                  + b_ref[...])


def _lin2_body(p0_ref, p1_ref, wt_ref, b_ref, o_ref):
    h = jax.nn.relu(p0_ref[...] + p1_ref[...])
    o_ref[...] = (jnp.dot(h, wt_ref[...],
                          preferred_element_type=jnp.float32)
                  + b_ref[...])


def _add_body(p0_ref, p1_ref, o_ref):
    o_ref[...] = p0_ref[...] + p1_ref[...]


def _row_spec():
    return pl.BlockSpec((_BM, D), lambda i: (i, 0))


def _full_spec(shape):
    return pl.BlockSpec(shape, lambda i: (0,) * len(shape))


_lin1 = pl.pallas_call(
    _lin1_body,
    grid=(N // _BM,),
    in_specs=[_row_spec(), _full_spec((D, D)), _full_spec((1, D))],
    out_specs=_row_spec(),
    out_shape=jax.ShapeDtypeStruct((N, D), jnp.float32),
)

_lin2 = pl.pallas_call(
    _lin2_body,
    grid=(N // _BM,),
    in_specs=[_row_spec(), _row_spec(), _full_spec((D, D)), _full_spec((1, D))],
    out_specs=_row_spec(),
    out_shape=jax.ShapeDtypeStruct((N, D), jnp.float32),
)

_add2 = pl.pallas_call(
    _add_body,
    grid=(N // _BM,),
    in_specs=[_row_spec(), _row_spec()],
    out_specs=_row_spec(),
    out_shape=jax.ShapeDtypeStruct((N, D), jnp.float32),
)


_PAD_IDX = None


def kernel(x, edge_index, edge_weight, W1, b1, W2, b2):
    pad = E2 - E
    pad_idx = (jnp.arange(pad, dtype=jnp.int32) * 7) % N
    dst = jnp.concatenate([edge_index[0], pad_idx])
    src = jnp.concatenate([edge_index[1], pad_idx])
    w = jnp.concatenate([edge_weight, jnp.zeros((pad,), jnp.float32)])
    h1 = _lin1(x, W1.T, b1.reshape(1, D))
    p = _spmm(h1, src, dst, w)
    h2 = _lin2(p[0], p[1], W2.T, b2.reshape(1, D))
    q = _spmm(h2, src, dst, w)
    return _add2(q[0], q[1])


# speedup vs baseline: 1.0940x; 1.0940x over previous
import functools

import jax
import jax.numpy as jnp
from jax import lax
from jax.experimental import pallas as pl
from jax.experimental.pallas import tpu as pltpu
from jax.experimental.pallas import tpu_sc as plsc

N = 10000
E = 320000
D = 128
L = 16
NC = 2
NS = 16
NW = NC * NS
K = 64
EPT = 10240
E2 = NW * EPT
NCH = EPT // K
NB = 4
RSTRIDE = 624
RPT = 640


def _lane_bcast(vec, k):
    idx = jnp.full((L, 1), k, dtype=jnp.int32)
    return lax.gather(
        vec, idx,
        lax.GatherDimensionNumbers(
            offset_dims=(), collapsed_slice_dims=(0,), start_index_map=(0,)),
        (1,), mode=lax.GatherScatterMode.PROMISE_IN_BOUNDS)


def _spmm_body(h_hbm, src_hbm, dst_hbm, w_hbm, out_hbm,
               src_v, w_bufs, dst_bufs, row_bufs, acc_sh,
               semg, semd, sems):
    c = lax.axis_index("c")
    s = lax.axis_index("s")
    wid = c * NS + s
    ebase = wid * EPT

    pltpu.sync_copy(src_hbm.at[pl.ds(ebase, EPT)], src_v)

    zero16 = jnp.zeros((L,), jnp.float32)
    def zfill(i, _):
        for j in range(D // L):
            row_bufs[0][i, pl.ds(j * L, L)] = zero16
            row_bufs[1][i, pl.ds(j * L, L)] = zero16
        return 0
    lax.fori_loop(0, K, zfill, 0)
    row0 = s * RSTRIDE
    nz = RPT // K
    for z in range(nz):
        pltpu.sync_copy(row_bufs[z % 2], acc_sh.at[pl.ds(row0 + z * K, K)])
    if RPT % K:
        pltpu.sync_copy(row_bufs[nz % 2],
                        acc_sh.at[pl.ds(row0 + RPT - K, K)])
    plsc.subcore_barrier()

    def issue(i, u):
        pltpu.async_copy(h_hbm.at[src_v.at[pl.ds(i * K, K)]],
                         row_bufs[u], semg[u])
        pltpu.async_copy(dst_hbm.at[pl.ds(ebase + i * K, K)],
                         dst_bufs[u], semd[u])
        pltpu.async_copy(w_hbm.at[pl.ds(ebase + i * K, K)],
                         w_bufs[u], semd[u])

    def mult(i, u):
        pltpu.make_async_copy(h_hbm.at[src_v.at[pl.ds(i * K, K)]],
                              row_bufs[u], semg[u]).wait()
        pltpu.make_async_copy(dst_hbm.at[pl.ds(ebase + i * K, K)],
                              dst_bufs[u], semd[u]).wait()
        pltpu.make_async_copy(w_hbm.at[pl.ds(ebase + i * K, K)],
                              w_bufs[u], semd[u]).wait()
        rbuf = row_bufs[u]
        wv = w_bufs[u]
        def group(g, _):
            wg = wv[pl.ds(g * L, L)]
            for k in range(L):
                wk = _lane_bcast(wg, k)
                r = g * L + k
                for j in range(D // L):
                    rbuf[r, pl.ds(j * L, L)] = rbuf[r, pl.ds(j * L, L)] * wk
            return 0
        lax.fori_loop(0, K // L, group, 0)

    def scatter_start(u):
        pltpu.async_copy(row_bufs[u], acc_sh.at[dst_bufs[u]], sems[u],
                         add=True)

    def scatter_wait(u):
        pltpu.make_async_copy(row_bufs[u], acc_sh.at[dst_bufs[u]],
                              sems[u]).wait()

    issue(0, 0)
    issue(1, 1)
    mult(0, 0)
    scatter_start(0)
    issue(2, 2)
    mult(1, 1)
    scatter_start(1)
    issue(3, 3)

    def body(t, _):
        i0 = 4 * t + 2
        for d in range(NB):
            i = i0 + d
            u = (2 + d) % NB
            mult(i, u)
            scatter_start(u)
            scatter_wait((u + 2) % NB)
            issue(i + 2, (u + 2) % NB)
        return 0

    lax.fori_loop(0, (NCH - 4) // 4, body, 0)

    i0 = NCH - 2
    mult(i0, i0 % NB)
    scatter_start(i0 % NB)
    scatter_wait((i0 + 2) % NB)
    mult(i0 + 1, (i0 + 1) % NB)
    scatter_start((i0 + 1) % NB)
    scatter_wait((i0 + 3) % NB)
    scatter_wait(i0 % NB)
    scatter_wait((i0 + 1) % NB)

    plsc.subcore_barrier()

    pltpu.sync_copy(acc_sh.at[pl.ds(row0, RPT)],
                    out_hbm.at[c, pl.ds(row0, RPT)])


_spmm = pl.kernel(
    _spmm_body,
    out_type=jax.ShapeDtypeStruct((NC, N, D), jnp.float32),
    mesh=plsc.VectorSubcoreMesh(core_axis_name="c", subcore_axis_name="s"),
    scratch_types=[
        pltpu.VMEM((EPT,), jnp.int32),
        [pltpu.VMEM((K,), jnp.float32) for _ in range(NB)],
        [pltpu.VMEM((K,), jnp.int32) for _ in range(NB)],
        [pltpu.VMEM((K, D), jnp.float32) for _ in range(NB)],
        pltpu.VMEM_SHARED((N, D), jnp.float32),
        [pltpu.SemaphoreType.DMA for _ in range(NB)],
        [pltpu.SemaphoreType.DMA for _ in range(NB)],
        [pltpu.SemaphoreType.DMA for _ in range(NB)],
    ],
)


_BM = 2000


def _lin1_body(x_ref, wt_ref, b_ref, o_ref):
    o_ref[...] = (jnp.dot(x_ref[...], wt_ref[...],
                          preferred_element_type=jnp.float32)
                  + b_ref[...])


def _lin2_body(p0_ref, p1_ref, wt_ref, b_ref, o_ref):
    h = jax.nn.relu(p0_ref[...] + p1_ref[...])
    o_ref[...] = (jnp.dot(h, wt_ref[...],
                          preferred_element_type=jnp.float32)
                  + b_ref[...])


def _add_body(p0_ref, p1_ref, o_ref):
    o_ref[...] = p0_ref[...] + p1_ref[...]


def _row_spec():
    return pl.BlockSpec((_BM, D), lambda i: (i, 0))


def _full_spec(shape):
    return pl.BlockSpec(shape, lambda i: (0,) * len(shape))


_lin1 = pl.pallas_call(
    _lin1_body,
    grid=(N // _BM,),
    in_specs=[_row_spec(), _full_spec((D, D)), _full_spec((1, D))],
    out_specs=_row_spec(),
    out_shape=jax.ShapeDtypeStruct((N, D), jnp.float32),
)

_lin2 = pl.pallas_call(
    _lin2_body,
    grid=(N // _BM,),
    in_specs=[_row_spec(), _row_spec(), _full_spec((D, D)), _full_spec((1, D))],
    out_specs=_row_spec(),
    out_shape=jax.ShapeDtypeStruct((N, D), jnp.float32),
)

_add2 = pl.pallas_call(
    _add_body,
    grid=(N // _BM,),
    in_specs=[_row_spec(), _row_spec()],
    out_specs=_row_spec(),
    out_shape=jax.ShapeDtypeStruct((N, D), jnp.float32),
)


_PAD_IDX = None


def kernel(x, edge_index, edge_weight, W1, b1, W2, b2):
    pad = E2 - E
    pad_idx = (jnp.arange(pad, dtype=jnp.int32) * 7) % N
    dst = jnp.concatenate([edge_index[0], pad_idx])
    src = jnp.concatenate([edge_index[1], pad_idx])
    w = jnp.concatenate([edge_weight, jnp.zeros((pad,), jnp.float32)])
    h1 = _lin1(x, W1.T, b1.reshape(1, D))
    p = _spmm(h1, src, dst, w)
    h2 = _lin2(p[0], p[1], W2.T, b2.reshape(1, D))
    q = _spmm(h2, src, dst, w)
    return _add2(q[0], q[1])
